# Initial kernel scaffold; baseline (speedup 1.0000x reference)
import jax
import jax.numpy as jnp
from jax.experimental import pallas as pl


def kernel(x, words, word_lengths, da_init):
    raise NotImplementedError("write your pallas kernel here")


# speedup vs baseline: 20.8457x; 20.8457x over previous
import functools

import jax
import jax.numpy as jnp
from jax import lax
from jax.experimental import pallas as pl
from jax.experimental.pallas import tpu as pltpu
from jax.experimental.pallas import tpu_sc as plsc

_NUM_WORDS = 1024
_MAX_LEN = 8
_DA_PAD = 32
_BS = 32
_NUM_CORES = 2
_NUM_SUBCORES = 16
_LANES = 16
_CHUNKS = _NUM_WORDS // _LANES
_D_SIZE = 10 * 10 * _LANES


def _dl_kernel(x_hbm, wt_hbm, wl_hbm, da0_hbm, out_hbm,
               x_v, wt_v, wl_v, da_v, d_v, out_v):
    wid = lax.axis_index("s") * _NUM_CORES + lax.axis_index("c")

    pltpu.sync_copy(x_hbm, x_v)
    pltpu.sync_copy(wt_hbm, wt_v)
    pltpu.sync_copy(wl_hbm, wl_v)
    pltpu.sync_copy(da0_hbm, da_v.at[pl.ds(0, _DA_PAD)])

    lane = lax.broadcasted_iota(jnp.int32, (_LANES,), 0)
    lane_f = lane.astype(jnp.float32)

    for t in range(1, _MAX_LEN + 1):
        col = (t - 2) % _MAX_LEN
        prev = da_v[pl.ds((t - 1) * _DA_PAD, _LANES)]
        da_v[pl.ds(t * _DA_PAD, _LANES)] = prev
        prev2 = da_v[pl.ds((t - 1) * _DA_PAD + _LANES, _LANES)]
        da_v[pl.ds(t * _DA_PAD + _LANES, _LANES)] = prev2
        tval = jnp.full((_LANES,), float(t), jnp.float32)
        for half in range(2):
            chars = plsc.load_gather(x_v, [(lane + half * _LANES) * _MAX_LEN + col])
            plsc.store_scatter(da_v, [t * _DA_PAD + chars], tval)

    base_x = wid * _MAX_LEN
    xrow = x_v[pl.ds(base_x, _LANES)]
    in_row = lane < _MAX_LEN
    sl_i = plsc.all_reduce_ffs(jnp.logical_and(xrow == 0, in_row))
    sl_i = jnp.broadcast_to(sl_i, (_LANES,)).astype(jnp.int32)
    sl_f = sl_i.astype(jnp.float32)

    xsplat = []
    for i in range(1, _MAX_LEN + 1):
        idx = jnp.broadcast_to(base_x + (i - 2) % _MAX_LEN, (_LANES,))
        xsplat.append(plsc.load_gather(x_v, [idx.astype(jnp.int32)]))

    col1 = []
    for q in range(_MAX_LEN):
        qv = jnp.full((_LANES,), q, jnp.int32)
        col1.append(jnp.where(qv < sl_i, float(q), 0.0).astype(jnp.float32))

    def chunk_body(c, carry):
        base = c * _LANES
        wl_c = wl_v[pl.ds(base * 1, _LANES)]
        wl_cf = wl_c.astype(jnp.float32)
        maxd = sl_f + wl_cf
        wcol = [wt_v[pl.ds(p * _NUM_WORDS + base, _LANES)]
                for p in range(_MAX_LEN)]

        for cc in range(10):
            d_v[pl.ds(cc * _LANES, _LANES)] = maxd
        for r in range(1, 10):
            d_v[pl.ds(r * 10 * _LANES, _LANES)] = maxd
        d_v[pl.ds((1 * 10 + 1) * _LANES, _LANES)] = jnp.zeros(
            (_LANES,), jnp.float32)
        for p in range(_MAX_LEN):
            val = jnp.where(wl_c > p, float(p), 0.0).astype(jnp.float32)
            d_v[pl.ds((1 * 10 + 2 + p) * _LANES, _LANES)] = val
        for q in range(_MAX_LEN):
            d_v[pl.ds(((2 + q) * 10 + 1) * _LANES, _LANES)] = col1[q]

        prevrow = [maxd] * (_MAX_LEN + 1)
        for i in range(1, _MAX_LEN + 1):
            xs = xsplat[i - 1]
            currow = [maxd] * (_MAX_LEN + 1)
            db = jnp.zeros((_LANES,), jnp.float32)
            for j in range(1, _MAX_LEN + 1):
                wc = wcol[(j - 2) % _MAX_LEN]
                da_row = (i - 1) if j == 1 else i
                k_f = plsc.load_gather(da_v, [da_row * _DA_PAD + wc])
                k_i = k_f.astype(jnp.int32)
                l_f = db
                l_i = db.astype(jnp.int32)
                d_t = plsc.load_gather(d_v, [(k_i * 10 + l_i) * _LANES + lane])
                eq = wc == xs
                cost = jnp.where(eq, 0.0, 1.0).astype(jnp.float32)
                db = jnp.where(eq, float(j), 0.0).astype(jnp.float32)
                cand4 = d_t + (float(i + j - 1) - k_f - l_f)
                up = prevrow[j] + 1.0
                left = (prevrow[0] if j == 1 else currow[j - 1]) + 1.0
                diag = prevrow[j - 1] + cost
                m = jnp.minimum(jnp.minimum(up, left),
                                jnp.minimum(diag, cand4))
                d_v[pl.ds((i * 10 + j) * _LANES, _LANES)] = m
                currow[j] = m
            prevrow = currow

        oidx = ((wl_c + 1) * 10 + (sl_i + 1)) * _LANES + lane
        out_v[pl.ds(base * 1, _LANES)] = plsc.load_gather(d_v, [oidx])
        return carry

    lax.fori_loop(0, _CHUNKS, chunk_body, 0, unroll=False)

    pltpu.sync_copy(out_v, out_hbm.at[wid])


@jax.jit
def kernel(x, words, word_lengths, da_init):
    bsz, seq, max_len = x.shape
    num_words = words.shape[0]
    mesh = plsc.VectorSubcoreMesh(core_axis_name="c", subcore_axis_name="s",
                                  num_cores=_NUM_CORES,
                                  num_subcores=_NUM_SUBCORES)

    x_flat = jnp.pad(x.reshape(-1), (0, _LANES)).astype(jnp.int32)
    wt = words.T.reshape(-1).astype(jnp.int32)
    da0 = jnp.pad(da_init, (0, _DA_PAD - da_init.shape[0])).astype(jnp.float32)

    run = pl.kernel(
        _dl_kernel,
        out_type=jax.ShapeDtypeStruct((_BS, _NUM_WORDS), jnp.float32),
        mesh=mesh,
        compiler_params=pltpu.CompilerParams(needs_layout_passes=False),
        scratch_types=[
            pltpu.VMEM((_BS * _MAX_LEN + _LANES,), jnp.int32),
            pltpu.VMEM((_MAX_LEN * _NUM_WORDS,), jnp.int32),
            pltpu.VMEM((_NUM_WORDS,), jnp.int32),
            pltpu.VMEM(((_MAX_LEN + 1) * _DA_PAD,), jnp.float32),
            pltpu.VMEM((_D_SIZE,), jnp.float32),
            pltpu.VMEM((_NUM_WORDS,), jnp.float32),
        ],
    )
    out = run(x_flat, wt, word_lengths.astype(jnp.int32), da0)
    return out.reshape(bsz, seq, num_words)


# speedup vs baseline: 31.5221x; 1.5122x over previous
import functools

import jax
import jax.numpy as jnp
from jax import lax
from jax.experimental import pallas as pl
from jax.experimental.pallas import tpu as pltpu
from jax.experimental.pallas import tpu_sc as plsc

_NUM_WORDS = 1024
_MAX_LEN = 8
_DA_PAD = 32
_BS = 32
_NUM_CORES = 2
_NUM_SUBCORES = 16
_LANES = 16
_CHUNKS = _NUM_WORDS // _LANES
_D_CELL = 10 * 10 * _LANES
_UNROLL = 4


def _dl_kernel(x_hbm, wt_hbm, wl_hbm, da0_hbm, out_hbm,
               x_v, wt_v, wl_v, da_v, d_v, out_v,
               sem_x, sem_wt, sem_wl, sem_da):
    wid = lax.axis_index("s") * _NUM_CORES + lax.axis_index("c")

    h_x = pltpu.async_copy(x_hbm, x_v, sem_x)
    h_wt = pltpu.async_copy(wt_hbm, wt_v, sem_wt)
    h_wl = pltpu.async_copy(wl_hbm, wl_v, sem_wl)
    h_da = pltpu.async_copy(da0_hbm, da_v.at[pl.ds(0, _DA_PAD)], sem_da)
    h_x.wait()
    h_da.wait()

    lane = lax.broadcasted_iota(jnp.int32, (_LANES,), 0)

    for t in range(1, _MAX_LEN + 1):
        col = (t - 2) % _MAX_LEN
        da_v[pl.ds(t * _DA_PAD, _LANES)] = da_v[pl.ds((t - 1) * _DA_PAD, _LANES)]
        da_v[pl.ds(t * _DA_PAD + _LANES, _LANES)] = (
            da_v[pl.ds((t - 1) * _DA_PAD + _LANES, _LANES)])
        tval = jnp.full((_LANES,), t, jnp.int32)
        for half in range(2):
            chars = plsc.load_gather(x_v, [(lane + half * _LANES) * _MAX_LEN + col])
            plsc.store_scatter(da_v, [t * _DA_PAD + chars], tval)

    base_x = wid * _MAX_LEN
    xrow = x_v[pl.ds(base_x, _LANES)]
    in_row = lane < _MAX_LEN
    sl_i = plsc.all_reduce_ffs(jnp.logical_and(xrow == 0, in_row))
    sl_i = jnp.broadcast_to(sl_i, (_LANES,)).astype(jnp.int32)
    sl_f = sl_i.astype(jnp.float32)

    xs, eq1, db1, mc1 = [], [], [], []
    one_f = jnp.ones((_LANES,), jnp.float32)
    for i in range(1, _MAX_LEN + 1):
        idx = jnp.broadcast_to(base_x + (i - 2) % _MAX_LEN, (_LANES,)).astype(
            jnp.int32)
        x_i = plsc.load_gather(x_v, [idx])
        e1 = x_i == 0
        k1 = plsc.load_gather(
            da_v, [jnp.full((_LANES,), (i - 1) * _DA_PAD, jnp.int32)])
        cost1 = jnp.where(e1, 0.0, 1.0).astype(jnp.float32)
        c4a = (i - k1).astype(jnp.float32)
        xs.append(x_i)
        eq1.append(e1)
        db1.append(jnp.where(e1, 1, 0).astype(jnp.int32))
        mc1.append(jnp.minimum(jnp.minimum(one_f, cost1), c4a))

    h_wt.wait()
    h_wl.wait()

    @plsc.parallel_loop(0, _CHUNKS, unroll=_UNROLL)
    def chunk_body(c):
        base = c * _LANES
        based = c * _D_CELL
        based_v = jnp.broadcast_to(based, (_LANES,)).astype(jnp.int32)
        wl_c = wl_v[pl.ds(base, _LANES)]
        maxd = sl_f + wl_c.astype(jnp.float32)
        wcol = [wt_v[pl.ds(p * _NUM_WORDS + base, _LANES)]
                for p in range(_MAX_LEN)]

        for cc in range(8):
            d_v[pl.ds(based + cc * _LANES, _LANES)] = maxd
        for r in range(1, 9):
            d_v[pl.ds(based + r * 10 * _LANES, _LANES)] = maxd

        prevrow = [maxd] * (_MAX_LEN + 1)
        for i in range(1, _MAX_LEN + 1):
            x_i = xs[i - 1]
            i_spl = jnp.full((_LANES,), i, jnp.int32)
            m = jnp.minimum(prevrow[1] + 1.0, maxd + mc1[i - 1])
            currow = [maxd, m] + [None] * (_MAX_LEN - 1)
            db_i = db1[i - 1]
            eq_prev = eq1[i - 1]
            for j in range(2, _MAX_LEN + 1):
                wc = wcol[j - 2]
                k_i = plsc.load_gather(da_v, [i * _DA_PAD + wc])
                l_i = db_i
                g = plsc.load_gather(
                    d_v, [based_v + k_i * 160 + l_i * _LANES + lane])
                reg = jnp.where(eq_prev, currow[j - 1], maxd)
                d_t = jnp.where(k_i == i_spl, reg, g)
                cand4 = d_t + (((i + j - 1) - k_i) - l_i).astype(jnp.float32)
                eq = wc == x_i
                cost = jnp.where(eq, 0.0, 1.0).astype(jnp.float32)
                db_i = jnp.where(eq, j, 0).astype(jnp.int32)
                eq_prev = eq
                m = jnp.minimum(
                    jnp.minimum(prevrow[j], currow[j - 1]) + 1.0,
                    jnp.minimum(prevrow[j - 1] + cost, cand4))
                currow[j] = m
            for j in range(1, _MAX_LEN + 1):
                d_v[pl.ds(based + (i * 10 + j) * _LANES, _LANES)] = currow[j]
            prevrow = currow

        oidx = based_v + ((wl_c + 1) * 10 + (sl_i + 1)) * _LANES + lane
        out_v[pl.ds(base, _LANES)] = plsc.load_gather(d_v, [oidx])

    pltpu.sync_copy(out_v, out_hbm.at[wid])


@jax.jit
def kernel(x, words, word_lengths, da_init):
    bsz, seq, max_len = x.shape
    num_words = words.shape[0]
    mesh = plsc.VectorSubcoreMesh(core_axis_name="c", subcore_axis_name="s",
                                  num_cores=_NUM_CORES,
                                  num_subcores=_NUM_SUBCORES)

    x_flat = jnp.pad(x.reshape(-1), (0, _LANES)).astype(jnp.int32)
    wt = words.T.reshape(-1).astype(jnp.int32)
    da0 = jnp.pad(da_init, (0, _DA_PAD - da_init.shape[0])).astype(jnp.int32)

    run = pl.kernel(
        _dl_kernel,
        out_type=jax.ShapeDtypeStruct((_BS, _NUM_WORDS), jnp.float32),
        mesh=mesh,
        compiler_params=pltpu.CompilerParams(needs_layout_passes=False),
        scratch_types=[
            pltpu.VMEM((_BS * _MAX_LEN + _LANES,), jnp.int32),
            pltpu.VMEM((_MAX_LEN * _NUM_WORDS,), jnp.int32),
            pltpu.VMEM((_NUM_WORDS,), jnp.int32),
            pltpu.VMEM(((_MAX_LEN + 1) * _DA_PAD,), jnp.int32),
            pltpu.VMEM((_CHUNKS * _D_CELL,), jnp.float32),
            pltpu.VMEM((_NUM_WORDS,), jnp.float32),
            pltpu.SemaphoreType.DMA,
            pltpu.SemaphoreType.DMA,
            pltpu.SemaphoreType.DMA,
            pltpu.SemaphoreType.DMA,
        ],
    )
    out = run(x_flat, wt, word_lengths.astype(jnp.int32), da0)
    return out.reshape(bsz, seq, num_words)
